# baseline (speedup 1.0000x reference)
import jax
import jax.numpy as jnp
from jax.experimental import pallas as pl


_BLK = 2048


def _mm_body(x_ref, wt_ref, o_ref):
    o_ref[...] = jnp.dot(x_ref[...], wt_ref[...],
                         preferred_element_type=jnp.float32)


def kernel(x, W):
    B, I = x.shape
    J = W.shape[0]
    wt = W.T
    grid = (B // _BLK,)
    return pl.pallas_call(
        _mm_body,
        grid=grid,
        in_specs=[
            pl.BlockSpec((_BLK, I), lambda g: (g, 0)),
            pl.BlockSpec((I, J), lambda g: (0, 0)),
        ],
        out_specs=pl.BlockSpec((_BLK, J), lambda g: (g, 0)),
        out_shape=jax.ShapeDtypeStruct((B, J), jnp.float32),
    )(x, wt)


# speedup vs baseline: 1.1007x; 1.1007x over previous
import jax
import jax.numpy as jnp
from jax.experimental import pallas as pl


_BLK = 16384


def _mm_body(x_ref, wt_ref, o_ref):
    o_ref[...] = jnp.dot(x_ref[...], wt_ref[...],
                         preferred_element_type=jnp.float32)


def kernel(x, W):
    B, I = x.shape
    J = W.shape[0]
    wt = W.T
    grid = (B // _BLK,)
    return pl.pallas_call(
        _mm_body,
        grid=grid,
        in_specs=[
            pl.BlockSpec((_BLK, I), lambda g: (g, 0)),
            pl.BlockSpec((I, J), lambda g: (0, 0)),
        ],
        out_specs=pl.BlockSpec((_BLK, J), lambda g: (g, 0)),
        out_shape=jax.ShapeDtypeStruct((B, J), jnp.float32),
    )(x, wt)


# speedup vs baseline: 3.7063x; 3.3673x over previous
import jax
import jax.numpy as jnp
from jax.experimental import pallas as pl


_BLK = 2048


def _mm_body(w_ref, xt_ref, o_ref):
    o_ref[...] = jnp.dot(w_ref[...], xt_ref[...],
                         preferred_element_type=jnp.float32)


def kernel(x, W):
    B, I = x.shape
    J = W.shape[0]
    xt = x.T
    outT = pl.pallas_call(
        _mm_body,
        grid=(B // _BLK,),
        in_specs=[
            pl.BlockSpec((J, I), lambda g: (0, 0)),
            pl.BlockSpec((I, _BLK), lambda g: (0, g)),
        ],
        out_specs=pl.BlockSpec((J, _BLK), lambda g: (0, g)),
        out_shape=jax.ShapeDtypeStruct((J, B), jnp.float32),
    )(W, xt)
    return outT.T
